# baseline (device time: 333472 ns/iter reference)
import jax
import jax.numpy as jnp
from jax import lax
from jax.experimental import pallas as pl
from jax.experimental.pallas import tpu as pltpu

N_DEV = 32
B_LOC = 2
SQ = 128
D = 512
H_LOC = 8
DH = 64
SCALE = 0.125

_MESH = pl.DeviceIdType.MESH


def kernel(x, Wq, Wo, Wk, Wv):
    bf16 = jnp.bfloat16
    xb = x.astype(bf16)
    wq = Wq.astype(bf16)
    wk = Wk.astype(bf16)
    wv = Wv.astype(bf16)
    wo = Wo.astype(bf16)

    def body(x_ref, wq_ref, wk_ref, wv_ref, wo_ref, out_ref,
             cxR, caR, cxL, caL, pcR, pcL, o_scr,
             sxsR, sxrR, sasR, sarR,
             sxsL, sxrL, sasL, sarL,
             xcredR, acredR, xcredL, acredL):
        my = lax.axis_index("i")
        left = lax.rem(my + N_DEV - 1, N_DEV)
        right = lax.rem(my + 1, N_DEV)

        barrier = pltpu.get_barrier_semaphore()
        pl.semaphore_signal(barrier, inc=1, device_id=(left,),
                            device_id_type=_MESH)
        pl.semaphore_signal(barrier, inc=1, device_id=(right,),
                            device_id_type=_MESH)
        pl.semaphore_wait(barrier, 2)

        def contribution(xc):
            q = jnp.dot(xc, wq_ref[...],
                        preferred_element_type=jnp.float32).astype(bf16)
            k = jnp.dot(xc, wk_ref[...],
                        preferred_element_type=jnp.float32).astype(bf16)
            v = jnp.dot(xc, wv_ref[...],
                        preferred_element_type=jnp.float32).astype(bf16)
            for h in range(H_LOC):
                cols = slice(h * DH, (h + 1) * DH)
                qs = q[:, cols]
                ks = k[:, cols]
                vs = v[:, cols]
                s = lax.dot_general(
                    qs, ks, (((1,), (1,)), ((), ())),
                    preferred_element_type=jnp.float32) * SCALE
                p = jnp.exp(s)
                l = jnp.sum(p, axis=1, keepdims=True)
                o = jnp.dot(p.astype(bf16), vs,
                            preferred_element_type=jnp.float32) / l
                o_scr[:, cols] = o.astype(bf16)
            return jnp.dot(o_scr[...], wo_ref[...],
                           preferred_element_type=jnp.float32)

        def fwd(buf, s, r, ssem, rsem, nbr):
            return pltpu.make_async_remote_copy(
                src_ref=buf.at[s], dst_ref=buf.at[r],
                send_sem=ssem.at[s], recv_sem=rsem.at[r],
                device_id=(nbr,), device_id_type=_MESH)

        def rcv(buf, r, ssem, rsem):
            return pltpu.make_async_remote_copy(
                src_ref=buf.at[r], dst_ref=buf.at[r],
                send_sem=ssem.at[r], recv_sem=rsem.at[r],
                device_id=(my,), device_id_type=_MESH)

        def cred_sig(cred, nbr):
            pl.semaphore_signal(cred, inc=1, device_id=(nbr,),
                                device_id_type=_MESH)

        def wait_sent(buf, s, ssem):
            fwd(buf, s, s, ssem, ssem, right).wait_send()

        cxR[0, :, :] = x_ref[0, :, :]
        cxL[0, :, :] = x_ref[1, :, :]
        fwd(cxR, 0, 1, sxsR, sxrR, right).start()
        fwd(cxL, 0, 1, sxsL, sxrL, left).start()
        caR[0, :, :] = contribution(cxR[0, :, :])
        fwd(caR, 0, 1, sasR, sarR, right).start()
        caL[0, :, :] = contribution(cxL[0, :, :])
        fwd(caL, 0, 1, sasL, sarL, left).start()
        rcv(cxR, 1, sxsR, sxrR).wait_recv()
        pcR[1, :, :] = contribution(cxR[1, :, :])
        rcv(cxL, 1, sxsL, sxrL).wait_recv()
        pcL[1, :, :] = contribution(cxL[1, :, :])
        wait_sent(cxR, 0, sxsR)
        cred_sig(xcredR, left)
        wait_sent(cxL, 0, sxsL)
        cred_sig(xcredL, right)
        pl.semaphore_wait(xcredR, 1)
        fwd(cxR, 1, 0, sxsR, sxrR, right).start()
        pl.semaphore_wait(xcredL, 1)
        fwd(cxL, 1, 0, sxsL, sxrL, left).start()

        def hop(rs, ns, last=False):
            wait_sent(caR, ns, sasR)
            cred_sig(acredR, left)
            wait_sent(caL, ns, sasL)
            cred_sig(acredL, right)
            rcv(caR, rs, sasR, sarR).wait_recv()
            caR[rs, :, :] = caR[rs, :, :] + pcR[rs, :, :]
            pl.semaphore_wait(acredR, 1)
            fwd(caR, rs, ns, sasR, sarR, right).start()
            rcv(caL, rs, sasL, sarL).wait_recv()
            caL[rs, :, :] = caL[rs, :, :] + pcL[rs, :, :]
            pl.semaphore_wait(acredL, 1)
            fwd(caL, rs, ns, sasL, sarL, left).start()

            wait_sent(cxR, rs, sxsR)
            cred_sig(xcredR, left)
            wait_sent(cxL, rs, sxsL)
            cred_sig(xcredL, right)
            if not last:
                rcv(cxR, ns, sxsR, sxrR).wait_recv()
                pl.semaphore_wait(xcredR, 1)
                fwd(cxR, ns, rs, sxsR, sxrR, right).start()
                rcv(cxL, ns, sxsL, sxrL).wait_recv()
                pl.semaphore_wait(xcredL, 1)
                fwd(cxL, ns, rs, sxsL, sxrL, left).start()
                pcR[ns, :, :] = contribution(cxR[ns, :, :])
                pcL[ns, :, :] = contribution(cxL[ns, :, :])

        def pair(i, carry):
            hop(1, 0)
            hop(0, 1)
            return carry

        lax.fori_loop(0, (N_DEV - 2) // 2, pair, 0)
        hop(1, 0, last=True)

        wait_sent(caR, 1, sasR)
        cred_sig(acredR, left)
        wait_sent(caL, 1, sasL)
        cred_sig(acredL, right)
        rcv(cxR, 0, sxsR, sxrR).wait_recv()
        rcv(cxL, 0, sxsL, sxrL).wait_recv()
        rcv(caR, 0, sasR, sarR).wait_recv()
        rcv(caL, 0, sasL, sarL).wait_recv()
        pl.semaphore_wait(xcredR, 1)
        pl.semaphore_wait(xcredL, 1)
        pl.semaphore_wait(acredR, 1)
        pl.semaphore_wait(acredL, 1)

        out_ref[0, :, :] = caR[0, :, :]
        out_ref[1, :, :] = caL[0, :, :]

    out = pl.pallas_call(
        body,
        out_shape=jax.ShapeDtypeStruct((B_LOC, SQ, D), jnp.float32),
        in_specs=[pl.BlockSpec(memory_space=pltpu.VMEM)] * 5,
        out_specs=pl.BlockSpec(memory_space=pltpu.VMEM),
        scratch_shapes=[
            pltpu.VMEM((2, SQ, D), bf16),
            pltpu.VMEM((2, SQ, D), jnp.float32),
            pltpu.VMEM((2, SQ, D), bf16),
            pltpu.VMEM((2, SQ, D), jnp.float32),
            pltpu.VMEM((2, SQ, D), jnp.float32),
            pltpu.VMEM((2, SQ, D), jnp.float32),
            pltpu.VMEM((SQ, D), bf16),
            pltpu.SemaphoreType.DMA((2,)),
            pltpu.SemaphoreType.DMA((2,)),
            pltpu.SemaphoreType.DMA((2,)),
            pltpu.SemaphoreType.DMA((2,)),
            pltpu.SemaphoreType.DMA((2,)),
            pltpu.SemaphoreType.DMA((2,)),
            pltpu.SemaphoreType.DMA((2,)),
            pltpu.SemaphoreType.DMA((2,)),
            pltpu.SemaphoreType.REGULAR,
            pltpu.SemaphoreType.REGULAR,
            pltpu.SemaphoreType.REGULAR,
            pltpu.SemaphoreType.REGULAR,
        ],
        compiler_params=pltpu.CompilerParams(collective_id=0),
    )(xb, wq, wk, wv, wo)
    return out


# device time: 299071 ns/iter; 1.1150x vs baseline; 1.1150x over previous
import jax
import jax.numpy as jnp
from jax import lax
from jax.experimental import pallas as pl
from jax.experimental.pallas import tpu as pltpu

N_DEV = 32
B_LOC = 2
SQ = 128
D = 512
H_LOC = 8
DH = 64
SCALE = 0.125

_MESH = pl.DeviceIdType.MESH


def kernel(x, Wq, Wo, Wk, Wv):
    bf16 = jnp.bfloat16
    xb = x.astype(bf16)
    wq = Wq.astype(bf16)
    wk = Wk.astype(bf16)
    wv = Wv.astype(bf16)
    wo = Wo.astype(bf16)

    def body(x_ref, wq_ref, wk_ref, wv_ref, wo_ref, out_ref,
             cxR, caR, cxL, caL, o_scr,
             sxsR, sxrR, sasR, sarR,
             sxsL, sxrL, sasL, sarL,
             xcredR, acredR, xcredL, acredL):
        my = lax.axis_index("i")
        left = lax.rem(my + N_DEV - 1, N_DEV)
        right = lax.rem(my + 1, N_DEV)

        barrier = pltpu.get_barrier_semaphore()
        pl.semaphore_signal(barrier, inc=1, device_id=(left,),
                            device_id_type=_MESH)
        pl.semaphore_signal(barrier, inc=1, device_id=(right,),
                            device_id_type=_MESH)
        pl.semaphore_wait(barrier, 2)

        def contribution(xc):
            q = jnp.dot(xc, wq_ref[...],
                        preferred_element_type=jnp.float32).astype(bf16)
            k = jnp.dot(xc, wk_ref[...],
                        preferred_element_type=jnp.float32).astype(bf16)
            v = jnp.dot(xc, wv_ref[...],
                        preferred_element_type=jnp.float32).astype(bf16)
            for h in range(H_LOC):
                cols = slice(h * DH, (h + 1) * DH)
                qs = q[:, cols]
                ks = k[:, cols]
                vs = v[:, cols]
                s = lax.dot_general(
                    qs, ks, (((1,), (1,)), ((), ())),
                    preferred_element_type=jnp.float32) * SCALE
                p = jnp.exp(s.astype(bf16))
                l = jnp.sum(p, axis=1, keepdims=True)
                o = jnp.dot(p, vs,
                            preferred_element_type=jnp.float32)
                o = o / l.astype(jnp.float32)
                o_scr[:, cols] = o.astype(bf16)
            return jnp.dot(o_scr[...], wo_ref[...],
                           preferred_element_type=jnp.float32)

        def fwd(buf, s, r, ssem, rsem, nbr):
            return pltpu.make_async_remote_copy(
                src_ref=buf.at[s], dst_ref=buf.at[r],
                send_sem=ssem.at[s], recv_sem=rsem.at[r],
                device_id=(nbr,), device_id_type=_MESH)

        def rcv(buf, r, ssem, rsem):
            return pltpu.make_async_remote_copy(
                src_ref=buf.at[r], dst_ref=buf.at[r],
                send_sem=ssem.at[r], recv_sem=rsem.at[r],
                device_id=(my,), device_id_type=_MESH)

        def cred_sig(cred, nbr):
            pl.semaphore_signal(cred, inc=1, device_id=(nbr,),
                                device_id_type=_MESH)

        def finish_prev_sends(s):
            fwd(cxR, s, s, sxsR, sxrR, right).wait_send()
            cred_sig(xcredR, left)
            fwd(cxL, s, s, sxsL, sxrL, left).wait_send()
            cred_sig(xcredL, right)
            fwd(caR, s, s, sasR, sarR, right).wait_send()
            cred_sig(acredR, left)
            fwd(caL, s, s, sasL, sarL, left).wait_send()
            cred_sig(acredL, right)

        cxR[0, :, :] = x_ref[0, :, :]
        cxL[0, :, :] = x_ref[1, :, :]
        fwd(cxR, 0, 1, sxsR, sxrR, right).start()
        fwd(cxL, 0, 1, sxsL, sxrL, left).start()
        caR[0, :, :] = contribution(cxR[0, :, :])
        fwd(caR, 0, 1, sasR, sarR, right).start()
        caL[0, :, :] = contribution(cxL[0, :, :])
        fwd(caL, 0, 1, sasL, sarL, left).start()

        def hop(rs, ns):
            finish_prev_sends(ns)

            rcv(cxR, rs, sxsR, sxrR).wait_recv()
            rcv(cxL, rs, sxsL, sxrL).wait_recv()
            pl.semaphore_wait(xcredR, 1)
            fwd(cxR, rs, ns, sxsR, sxrR, right).start()
            pl.semaphore_wait(xcredL, 1)
            fwd(cxL, rs, ns, sxsL, sxrL, left).start()

            cR = contribution(cxR[rs, :, :])
            rcv(caR, rs, sasR, sarR).wait_recv()
            caR[rs, :, :] = caR[rs, :, :] + cR
            pl.semaphore_wait(acredR, 1)
            fwd(caR, rs, ns, sasR, sarR, right).start()

            cL = contribution(cxL[rs, :, :])
            rcv(caL, rs, sasL, sarL).wait_recv()
            caL[rs, :, :] = caL[rs, :, :] + cL
            pl.semaphore_wait(acredL, 1)
            fwd(caL, rs, ns, sasL, sarL, left).start()

        def pair(i, carry):
            hop(1, 0)
            hop(0, 1)
            return carry

        lax.fori_loop(0, (N_DEV - 2) // 2, pair, 0)
        hop(1, 0)

        finish_prev_sends(1)
        rcv(cxR, 0, sxsR, sxrR).wait_recv()
        rcv(cxL, 0, sxsL, sxrL).wait_recv()
        rcv(caR, 0, sasR, sarR).wait_recv()
        rcv(caL, 0, sasL, sarL).wait_recv()
        pl.semaphore_wait(xcredR, 1)
        pl.semaphore_wait(xcredL, 1)
        pl.semaphore_wait(acredR, 1)
        pl.semaphore_wait(acredL, 1)

        out_ref[0, :, :] = caR[0, :, :]
        out_ref[1, :, :] = caL[0, :, :]

    out = pl.pallas_call(
        body,
        out_shape=jax.ShapeDtypeStruct((B_LOC, SQ, D), jnp.float32),
        in_specs=[pl.BlockSpec(memory_space=pltpu.VMEM)] * 5,
        out_specs=pl.BlockSpec(memory_space=pltpu.VMEM),
        scratch_shapes=[
            pltpu.VMEM((2, SQ, D), bf16),
            pltpu.VMEM((2, SQ, D), jnp.float32),
            pltpu.VMEM((2, SQ, D), bf16),
            pltpu.VMEM((2, SQ, D), jnp.float32),
            pltpu.VMEM((SQ, D), bf16),
            pltpu.SemaphoreType.DMA((2,)),
            pltpu.SemaphoreType.DMA((2,)),
            pltpu.SemaphoreType.DMA((2,)),
            pltpu.SemaphoreType.DMA((2,)),
            pltpu.SemaphoreType.DMA((2,)),
            pltpu.SemaphoreType.DMA((2,)),
            pltpu.SemaphoreType.DMA((2,)),
            pltpu.SemaphoreType.DMA((2,)),
            pltpu.SemaphoreType.REGULAR,
            pltpu.SemaphoreType.REGULAR,
            pltpu.SemaphoreType.REGULAR,
            pltpu.SemaphoreType.REGULAR,
        ],
        compiler_params=pltpu.CompilerParams(collective_id=0),
    )(xb, wq, wk, wv, wo)
    return out


# device time: 243191 ns/iter; 1.3712x vs baseline; 1.2298x over previous
import jax
import jax.numpy as jnp
from jax import lax
from jax.experimental import pallas as pl
from jax.experimental.pallas import tpu as pltpu

N_DEV = 32
B_LOC = 2
SQ = 128
D = 512
H_LOC = 8
DH = 64
SCALE = 0.125

_MESH = pl.DeviceIdType.MESH


def kernel(x, Wq, Wo, Wk, Wv):
    bf16 = jnp.bfloat16
    xb = x.astype(bf16)
    wq = Wq.astype(bf16)
    wk = Wk.astype(bf16)
    wv = Wv.astype(bf16)
    wo = Wo.astype(bf16)

    def body(x_ref, wq_ref, wk_ref, wv_ref, wo_ref, out_ref,
             cxR, caR, cxL, caL, o_scr,
             sxsR, sxrR, sasR, sarR,
             sxsL, sxrL, sasL, sarL,
             xcredR, acredR, xcredL, acredL):
        my = lax.axis_index("i")
        left = lax.rem(my + N_DEV - 1, N_DEV)
        right = lax.rem(my + 1, N_DEV)

        barrier = pltpu.get_barrier_semaphore()
        pl.semaphore_signal(barrier, inc=1, device_id=(left,),
                            device_id_type=_MESH)
        pl.semaphore_signal(barrier, inc=1, device_id=(right,),
                            device_id_type=_MESH)
        pl.semaphore_wait(barrier, 2)

        def contribution(xc):
            q = jnp.dot(xc, wq_ref[...],
                        preferred_element_type=jnp.float32).astype(bf16)
            k = jnp.dot(xc, wk_ref[...],
                        preferred_element_type=jnp.float32).astype(bf16)
            v = jnp.dot(xc, wv_ref[...],
                        preferred_element_type=jnp.float32).astype(bf16)
            for h in range(H_LOC):
                cols = slice(h * DH, (h + 1) * DH)
                qs = q[:, cols]
                ks = k[:, cols]
                vs = v[:, cols]
                s = lax.dot_general(
                    qs, ks, (((1,), (1,)), ((), ())),
                    preferred_element_type=jnp.float32) * SCALE
                p = jnp.exp(s.astype(bf16))
                l = jnp.sum(p, axis=1, keepdims=True)
                o = jnp.dot(p, vs,
                            preferred_element_type=jnp.float32)
                o = o / l.astype(jnp.float32)
                o_scr[:, cols] = o.astype(bf16)
            return jnp.dot(o_scr[...], wo_ref[...],
                           preferred_element_type=jnp.float32)

        def fwd(buf, s, r, ssem, rsem, nbr):
            return pltpu.make_async_remote_copy(
                src_ref=buf.at[s], dst_ref=buf.at[r],
                send_sem=ssem.at[s], recv_sem=rsem.at[r],
                device_id=(nbr,), device_id_type=_MESH)

        def rcv(buf, r, ssem, rsem):
            return pltpu.make_async_remote_copy(
                src_ref=buf.at[r], dst_ref=buf.at[r],
                send_sem=ssem.at[r], recv_sem=rsem.at[r],
                device_id=(my,), device_id_type=_MESH)

        def cred_sig(cred, nbr):
            pl.semaphore_signal(cred, inc=1, device_id=(nbr,),
                                device_id_type=_MESH)

        def finish_prev_sends(s):
            fwd(cxR, s, s, sxsR, sxrR, right).wait_send()
            cred_sig(xcredR, left)
            fwd(cxL, s, s, sxsL, sxrL, left).wait_send()
            cred_sig(xcredL, right)
            fwd(caR, s, s, sasR, sarR, right).wait_send()
            cred_sig(acredR, left)
            fwd(caL, s, s, sasL, sarL, left).wait_send()
            cred_sig(acredL, right)

        cxR[0, :, :] = x_ref[0, :, :]
        cxL[0, :, :] = x_ref[1, :, :]
        fwd(cxR, 0, 1, sxsR, sxrR, right).start()
        fwd(cxL, 0, 1, sxsL, sxrL, left).start()
        caR[0, :, :] = contribution(cxR[0, :, :]).astype(bf16)
        fwd(caR, 0, 1, sasR, sarR, right).start()
        caL[0, :, :] = contribution(cxL[0, :, :]).astype(bf16)
        fwd(caL, 0, 1, sasL, sarL, left).start()

        def hop(rs, ns):
            finish_prev_sends(ns)

            rcv(cxR, rs, sxsR, sxrR).wait_recv()
            rcv(cxL, rs, sxsL, sxrL).wait_recv()
            pl.semaphore_wait(xcredR, 1)
            fwd(cxR, rs, ns, sxsR, sxrR, right).start()
            pl.semaphore_wait(xcredL, 1)
            fwd(cxL, rs, ns, sxsL, sxrL, left).start()

            cR = contribution(cxR[rs, :, :])
            rcv(caR, rs, sasR, sarR).wait_recv()
            caR[rs, :, :] = (caR[rs, :, :] + cR).astype(bf16)
            pl.semaphore_wait(acredR, 1)
            fwd(caR, rs, ns, sasR, sarR, right).start()

            cL = contribution(cxL[rs, :, :])
            rcv(caL, rs, sasL, sarL).wait_recv()
            caL[rs, :, :] = (caL[rs, :, :] + cL).astype(bf16)
            pl.semaphore_wait(acredL, 1)
            fwd(caL, rs, ns, sasL, sarL, left).start()

        def pair(i, carry):
            hop(1, 0)
            hop(0, 1)
            return carry

        lax.fori_loop(0, (N_DEV - 2) // 2, pair, 0)
        hop(1, 0)

        finish_prev_sends(1)
        rcv(cxR, 0, sxsR, sxrR).wait_recv()
        rcv(cxL, 0, sxsL, sxrL).wait_recv()
        rcv(caR, 0, sasR, sarR).wait_recv()
        rcv(caL, 0, sasL, sarL).wait_recv()
        pl.semaphore_wait(xcredR, 1)
        pl.semaphore_wait(xcredL, 1)
        pl.semaphore_wait(acredR, 1)
        pl.semaphore_wait(acredL, 1)

        out_ref[0, :, :] = caR[0, :, :].astype(jnp.float32)
        out_ref[1, :, :] = caL[0, :, :].astype(jnp.float32)

    out = pl.pallas_call(
        body,
        out_shape=jax.ShapeDtypeStruct((B_LOC, SQ, D), jnp.float32),
        in_specs=[pl.BlockSpec(memory_space=pltpu.VMEM)] * 5,
        out_specs=pl.BlockSpec(memory_space=pltpu.VMEM),
        scratch_shapes=[
            pltpu.VMEM((2, SQ, D), bf16),
            pltpu.VMEM((2, SQ, D), bf16),
            pltpu.VMEM((2, SQ, D), bf16),
            pltpu.VMEM((2, SQ, D), bf16),
            pltpu.VMEM((SQ, D), bf16),
            pltpu.SemaphoreType.DMA((2,)),
            pltpu.SemaphoreType.DMA((2,)),
            pltpu.SemaphoreType.DMA((2,)),
            pltpu.SemaphoreType.DMA((2,)),
            pltpu.SemaphoreType.DMA((2,)),
            pltpu.SemaphoreType.DMA((2,)),
            pltpu.SemaphoreType.DMA((2,)),
            pltpu.SemaphoreType.DMA((2,)),
            pltpu.SemaphoreType.REGULAR,
            pltpu.SemaphoreType.REGULAR,
            pltpu.SemaphoreType.REGULAR,
            pltpu.SemaphoreType.REGULAR,
        ],
        compiler_params=pltpu.CompilerParams(collective_id=0),
    )(xb, wq, wk, wv, wo)
    return out
